# baseline (device time: 353953 ns/iter reference)
import jax
import jax.numpy as jnp
from jax import lax
from jax.experimental import pallas as pl
from jax.experimental.pallas import tpu as pltpu

N_DEV = 32
B_LOC = 256
D = 256
B_GLOB = N_DEV * B_LOC
N_PHASES = 6
CHUNK = 2048


def kernel(x, Win0, Wout0, Win1, Wout1, Win2, Wout2):
    def body(x_ref, win0_ref, wout0_ref, win1_ref, wout1_ref, win2_ref,
             wout2_ref, out_ref, xfull, partial, rs_buf, blk,
             send_sems, recv_sems):
        me = lax.axis_index("i")

        barrier_sem = pltpu.get_barrier_semaphore()

        def bar_k(k, c):
            dst = lax.rem(me + k, N_DEV)
            pl.semaphore_signal(
                barrier_sem, inc=1,
                device_id=(dst,), device_id_type=pl.DeviceIdType.MESH,
            )
            return c
        lax.fori_loop(1, N_DEV, bar_k, 0)
        pl.semaphore_wait(barrier_sem, N_DEV - 1)

        def flat_exchange(phase, src_for, dst_slice):
            def send_k(k, c):
                dst = lax.rem(me + k, N_DEV)
                pltpu.make_async_remote_copy(
                    src_ref=src_for(dst),
                    dst_ref=dst_slice(me),
                    send_sem=send_sems.at[k],
                    recv_sem=recv_sems.at[phase, me],
                    device_id=(dst,),
                    device_id_type=pl.DeviceIdType.MESH,
                ).start()
                return c
            lax.fori_loop(1, N_DEV, send_k, 0)

            def wait_r(k, c):
                src = lax.rem(me + k, N_DEV)
                pltpu.make_async_remote_copy(
                    src_ref=src_for(src),
                    dst_ref=dst_slice(src),
                    send_sem=send_sems.at[k],
                    recv_sem=recv_sems.at[phase, src],
                    device_id=(src,),
                    device_id_type=pl.DeviceIdType.MESH,
                ).wait_recv()
                return c
            lax.fori_loop(1, N_DEV, wait_r, 0)

            def wait_s(k, c):
                dst = lax.rem(me + k, N_DEV)
                pltpu.make_async_remote_copy(
                    src_ref=src_for(dst),
                    dst_ref=dst_slice(me),
                    send_sem=send_sems.at[k],
                    recv_sem=recv_sems.at[phase, me],
                    device_id=(dst,),
                    device_id_type=pl.DeviceIdType.MESH,
                ).wait_send()
                return c
            lax.fori_loop(1, N_DEV, wait_s, 0)

        def row(ref, idx):
            return ref.at[pl.ds(idx * B_LOC, B_LOC), :]

        def all_gather(phase):
            xfull[pl.ds(me * B_LOC, B_LOC), :] = blk[...]
            flat_exchange(phase, lambda dst: blk, lambda src: row(xfull, src))

        def reduce_scatter(phase):
            rs_buf[pl.ds(me * B_LOC, B_LOC), :] = partial[
                pl.ds(me * B_LOC, B_LOC), :]
            flat_exchange(
                phase, lambda dst: row(partial, dst),
                lambda src: row(rs_buf, src))

        def reduce_sum():
            return jnp.sum(
                rs_buf[...].reshape(N_DEV, B_LOC, D).astype(jnp.float32),
                axis=0)

        def layer(win_ref, wout_ref):
            win = win_ref[...].astype(jnp.bfloat16)
            wout = wout_ref[...].astype(jnp.bfloat16)
            for c in range(0, B_GLOB, CHUNK):
                xc = xfull[pl.ds(c, CHUNK), :]
                h = jnp.maximum(
                    jnp.dot(xc, win, preferred_element_type=jnp.float32),
                    0.0)
                p = jnp.dot(h.astype(jnp.bfloat16), wout,
                            preferred_element_type=jnp.float32)
                partial[pl.ds(c, CHUNK), :] = p.astype(jnp.bfloat16)

        blk[...] = x_ref[...].astype(jnp.bfloat16)
        all_gather(0)

        for l, (win_ref, wout_ref) in enumerate(
                [(win0_ref, wout0_ref), (win1_ref, wout1_ref),
                 (win2_ref, wout2_ref)]):
            layer(win_ref, wout_ref)
            reduce_scatter(2 * l + 1)
            acc = reduce_sum()
            if l < 2:
                blk[...] = acc.astype(jnp.bfloat16)
                all_gather(2 * l + 2)
            else:
                out_ref[...] = acc

    return pl.pallas_call(
        body,
        out_shape=jax.ShapeDtypeStruct((B_LOC, D), jnp.float32),
        in_specs=[pl.BlockSpec(memory_space=pltpu.VMEM)] * 7,
        out_specs=pl.BlockSpec(memory_space=pltpu.VMEM),
        scratch_shapes=[
            pltpu.VMEM((B_GLOB, D), jnp.bfloat16),
            pltpu.VMEM((B_GLOB, D), jnp.bfloat16),
            pltpu.VMEM((B_GLOB, D), jnp.bfloat16),
            pltpu.VMEM((B_LOC, D), jnp.bfloat16),
            pltpu.SemaphoreType.DMA((N_DEV,)),
            pltpu.SemaphoreType.DMA((N_PHASES, N_DEV)),
        ],
        compiler_params=pltpu.CompilerParams(collective_id=0),
    )(x, Win0, Wout0, Win1, Wout1, Win2, Wout2)


# device time: 341178 ns/iter; 1.0374x vs baseline; 1.0374x over previous
import jax
import jax.numpy as jnp
from jax import lax
from jax.experimental import pallas as pl
from jax.experimental.pallas import tpu as pltpu

N_DEV = 32
B_LOC = 256
D = 256
B_GLOB = N_DEV * B_LOC


def kernel(x, Win0, Wout0, Win1, Wout1, Win2, Wout2):
    def body(x_ref, win0_ref, wout0_ref, win1_ref, wout1_ref, win2_ref,
             wout2_ref, out_ref, xfull, partial, rs_buf, blk, acc,
             send_sems, recv_sems):
        me = lax.axis_index("i")

        barrier_sem = pltpu.get_barrier_semaphore()

        def bar_k(k, c):
            dst = lax.rem(me + k, N_DEV)
            pl.semaphore_signal(
                barrier_sem, inc=1,
                device_id=(dst,), device_id_type=pl.DeviceIdType.MESH,
            )
            return c
        lax.fori_loop(1, N_DEV, bar_k, 0)
        pl.semaphore_wait(barrier_sem, N_DEV - 1)

        def rowblock(ref, idx):
            return ref.at[pl.ds(idx * B_LOC, B_LOC), :]

        blk[...] = x_ref[...].astype(jnp.bfloat16)

        for l, (win_ref, wout_ref) in enumerate(
                [(win0_ref, wout0_ref), (win1_ref, wout1_ref),
                 (win2_ref, wout2_ref)]):
            xphase, pphase = 2 * l, 2 * l + 1
            win = win_ref[...].astype(jnp.bfloat16)
            wout = wout_ref[...].astype(jnp.bfloat16)

            def f_of(xb):
                h = jnp.maximum(
                    jnp.dot(xb, win, preferred_element_type=jnp.float32),
                    0.0)
                return jnp.dot(h.astype(jnp.bfloat16), wout,
                               preferred_element_type=jnp.float32)

            def ag_send(k, c):
                dst = lax.rem(me + k, N_DEV)
                pltpu.make_async_remote_copy(
                    src_ref=blk,
                    dst_ref=rowblock(xfull, me),
                    send_sem=send_sems.at[0, k],
                    recv_sem=recv_sems.at[xphase, me],
                    device_id=(dst,),
                    device_id_type=pl.DeviceIdType.MESH,
                ).start()
                return c
            lax.fori_loop(1, N_DEV, ag_send, 0)

            acc[...] = f_of(blk[...])

            def block_step(k, c):
                src = lax.rem(me + k, N_DEV)
                pltpu.make_async_remote_copy(
                    src_ref=blk,
                    dst_ref=rowblock(xfull, src),
                    send_sem=send_sems.at[0, k],
                    recv_sem=recv_sems.at[xphase, src],
                    device_id=(src,),
                    device_id_type=pl.DeviceIdType.MESH,
                ).wait_recv()
                partial[pl.ds(src * B_LOC, B_LOC), :] = (
                    f_of(xfull[pl.ds(src * B_LOC, B_LOC), :])
                    .astype(jnp.bfloat16))
                pltpu.make_async_remote_copy(
                    src_ref=rowblock(partial, src),
                    dst_ref=rowblock(rs_buf, me),
                    send_sem=send_sems.at[1, k],
                    recv_sem=recv_sems.at[pphase, me],
                    device_id=(src,),
                    device_id_type=pl.DeviceIdType.MESH,
                ).start()
                return c
            lax.fori_loop(1, N_DEV, block_step, 0)

            def acc_step(k, c):
                src = lax.rem(me + k, N_DEV)
                pltpu.make_async_remote_copy(
                    src_ref=rowblock(partial, src),
                    dst_ref=rowblock(rs_buf, src),
                    send_sem=send_sems.at[1, k],
                    recv_sem=recv_sems.at[pphase, src],
                    device_id=(src,),
                    device_id_type=pl.DeviceIdType.MESH,
                ).wait_recv()
                acc[...] = acc[...] + rs_buf[
                    pl.ds(src * B_LOC, B_LOC), :].astype(jnp.float32)
                return c
            lax.fori_loop(1, N_DEV, acc_step, 0)

            def send_wait(k, c):
                dst = lax.rem(me + k, N_DEV)
                pltpu.make_async_remote_copy(
                    src_ref=blk,
                    dst_ref=rowblock(xfull, me),
                    send_sem=send_sems.at[0, k],
                    recv_sem=recv_sems.at[xphase, me],
                    device_id=(dst,),
                    device_id_type=pl.DeviceIdType.MESH,
                ).wait_send()
                pltpu.make_async_remote_copy(
                    src_ref=rowblock(partial, dst),
                    dst_ref=rowblock(rs_buf, me),
                    send_sem=send_sems.at[1, k],
                    recv_sem=recv_sems.at[pphase, me],
                    device_id=(dst,),
                    device_id_type=pl.DeviceIdType.MESH,
                ).wait_send()
                return c
            lax.fori_loop(1, N_DEV, send_wait, 0)

            if l < 2:
                blk[...] = acc[...].astype(jnp.bfloat16)
            else:
                out_ref[...] = acc[...]

    return pl.pallas_call(
        body,
        out_shape=jax.ShapeDtypeStruct((B_LOC, D), jnp.float32),
        in_specs=[pl.BlockSpec(memory_space=pltpu.VMEM)] * 7,
        out_specs=pl.BlockSpec(memory_space=pltpu.VMEM),
        scratch_shapes=[
            pltpu.VMEM((B_GLOB, D), jnp.bfloat16),
            pltpu.VMEM((B_GLOB, D), jnp.bfloat16),
            pltpu.VMEM((B_GLOB, D), jnp.bfloat16),
            pltpu.VMEM((B_LOC, D), jnp.bfloat16),
            pltpu.VMEM((B_LOC, D), jnp.float32),
            pltpu.SemaphoreType.DMA((2, N_DEV)),
            pltpu.SemaphoreType.DMA((6, N_DEV)),
        ],
        compiler_params=pltpu.CompilerParams(collective_id=0),
    )(x, Win0, Wout0, Win1, Wout1, Win2, Wout2)


# device time: 339343 ns/iter; 1.0431x vs baseline; 1.0054x over previous
import jax
import jax.numpy as jnp
from jax import lax
from jax.experimental import pallas as pl
from jax.experimental.pallas import tpu as pltpu

N_DEV = 32
B_LOC = 256
D = 256
B_GLOB = N_DEV * B_LOC
G = 8
N_GRP = N_DEV // G


def kernel(x, Win0, Wout0, Win1, Wout1, Win2, Wout2):
    def body(x_ref, win0_ref, wout0_ref, win1_ref, wout1_ref, win2_ref,
             wout2_ref, out_ref, xfull, partial, rs_buf, blk, acc,
             send_sems, recv_sems):
        me = lax.axis_index("i")

        barrier_sem = pltpu.get_barrier_semaphore()

        def bar_k(k, c):
            dst = lax.rem(me + k, N_DEV)
            pl.semaphore_signal(
                barrier_sem, inc=1,
                device_id=(dst,), device_id_type=pl.DeviceIdType.MESH,
            )
            return c
        lax.fori_loop(1, N_DEV, bar_k, 0)
        pl.semaphore_wait(barrier_sem, N_DEV - 1)

        def rel(ref, r):
            return ref.at[pl.ds(r * B_LOC, B_LOC), :]

        blk[...] = x_ref[...].astype(jnp.bfloat16)

        for l, (win_ref, wout_ref) in enumerate(
                [(win0_ref, wout0_ref), (win1_ref, wout1_ref),
                 (win2_ref, wout2_ref)]):
            xphase, pphase = 2 * l, 2 * l + 1
            win = win_ref[...].astype(jnp.bfloat16)
            wout = wout_ref[...].astype(jnp.bfloat16)

            def f_of(xb):
                h = jnp.maximum(
                    jnp.dot(xb, win, preferred_element_type=jnp.float32),
                    0.0)
                return jnp.dot(h.astype(jnp.bfloat16), wout,
                               preferred_element_type=jnp.float32)

            xfull[0:B_LOC, :] = blk[...]

            def ag_send(k, c):
                dst = lax.rem(me + k, N_DEV)
                r = N_DEV - k
                pltpu.make_async_remote_copy(
                    src_ref=blk,
                    dst_ref=rel(xfull, r),
                    send_sem=send_sems.at[0, k],
                    recv_sem=recv_sems.at[xphase, r],
                    device_id=(dst,),
                    device_id_type=pl.DeviceIdType.MESH,
                ).start()
                return c
            lax.fori_loop(1, N_DEV, ag_send, 0)

            for g in range(N_GRP - 1, -1, -1):
                lo = g * G
                lo_w = max(lo, 1)

                def wait_x(r, c):
                    pltpu.make_async_remote_copy(
                        src_ref=blk,
                        dst_ref=rel(xfull, r),
                        send_sem=send_sems.at[0, 1],
                        recv_sem=recv_sems.at[xphase, r],
                        device_id=(me,),
                        device_id_type=pl.DeviceIdType.MESH,
                    ).wait_recv()
                    return c
                lax.fori_loop(lo_w, lo + G, wait_x, 0)

                P = f_of(xfull[pl.ds(lo * B_LOC, G * B_LOC), :])
                partial[pl.ds(lo * B_LOC, G * B_LOC), :] = (
                    P.astype(jnp.bfloat16))
                if g == 0:
                    acc[...] = P[0:B_LOC]

                def rs_send(r, c):
                    dst = lax.rem(me + r, N_DEV)
                    rr = N_DEV - r
                    pltpu.make_async_remote_copy(
                        src_ref=rel(partial, r),
                        dst_ref=rel(rs_buf, rr),
                        send_sem=send_sems.at[1, r],
                        recv_sem=recv_sems.at[pphase, rr],
                        device_id=(dst,),
                        device_id_type=pl.DeviceIdType.MESH,
                    ).start()
                    return c
                lax.fori_loop(lo_w, lo + G, rs_send, 0)

            for g in range(N_GRP):
                lo = g * G
                lo_w = max(lo, 1)
                cnt = lo + G - lo_w

                def wait_p(k, c):
                    pltpu.make_async_remote_copy(
                        src_ref=blk,
                        dst_ref=rel(rs_buf, k),
                        send_sem=send_sems.at[1, 1],
                        recv_sem=recv_sems.at[pphase, k],
                        device_id=(me,),
                        device_id_type=pl.DeviceIdType.MESH,
                    ).wait_recv()
                    return c
                lax.fori_loop(lo_w, lo + G, wait_p, 0)

                acc[...] = acc[...] + jnp.sum(
                    rs_buf[pl.ds(lo_w * B_LOC, cnt * B_LOC), :]
                    .reshape(cnt, B_LOC, D).astype(jnp.float32),
                    axis=0)

            def send_wait(k, c):
                dst = lax.rem(me + k, N_DEV)
                pltpu.make_async_remote_copy(
                    src_ref=blk,
                    dst_ref=rel(xfull, 1),
                    send_sem=send_sems.at[0, k],
                    recv_sem=recv_sems.at[xphase, 1],
                    device_id=(dst,),
                    device_id_type=pl.DeviceIdType.MESH,
                ).wait_send()
                pltpu.make_async_remote_copy(
                    src_ref=rel(partial, k),
                    dst_ref=rel(rs_buf, 1),
                    send_sem=send_sems.at[1, k],
                    recv_sem=recv_sems.at[pphase, 1],
                    device_id=(dst,),
                    device_id_type=pl.DeviceIdType.MESH,
                ).wait_send()
                return c
            lax.fori_loop(1, N_DEV, send_wait, 0)

            if l < 2:
                blk[...] = acc[...].astype(jnp.bfloat16)
            else:
                out_ref[...] = acc[...]

    return pl.pallas_call(
        body,
        out_shape=jax.ShapeDtypeStruct((B_LOC, D), jnp.float32),
        in_specs=[pl.BlockSpec(memory_space=pltpu.VMEM)] * 7,
        out_specs=pl.BlockSpec(memory_space=pltpu.VMEM),
        scratch_shapes=[
            pltpu.VMEM((B_GLOB, D), jnp.bfloat16),
            pltpu.VMEM((B_GLOB, D), jnp.bfloat16),
            pltpu.VMEM((B_GLOB, D), jnp.bfloat16),
            pltpu.VMEM((B_LOC, D), jnp.bfloat16),
            pltpu.VMEM((B_LOC, D), jnp.float32),
            pltpu.SemaphoreType.DMA((2, N_DEV)),
            pltpu.SemaphoreType.DMA((6, N_DEV)),
        ],
        compiler_params=pltpu.CompilerParams(collective_id=0),
    )(x, Win0, Wout0, Win1, Wout1, Win2, Wout2)
